# baseline (speedup 1.0000x reference)
import functools

import jax
import jax.numpy as jnp
from jax import lax
from jax.experimental import pallas as pl
from jax.experimental.pallas import tpu as pltpu
from jax.experimental.pallas import tpu_sc as plsc

D = 64
B = 4096
NF = 5

_info = plsc.get_sparse_core_info()
_NC = _info.num_cores
_NS = _info.num_subcores
_NW = _NC * _NS
_BPW = B // _NW

_mesh = plsc.VectorSubcoreMesh(core_axis_name="c", subcore_axis_name="s")


@functools.partial(
    pl.kernel,
    mesh=_mesh,
    out_type=jax.ShapeDtypeStruct((NF, B, D), jnp.float32),
    scratch_types=(
        [pltpu.VMEM((_BPW,), jnp.int32) for _ in range(NF)]
        + [pltpu.VMEM((_BPW, D), jnp.float32) for _ in range(NF)]
        + [pltpu.SemaphoreType.DMA for _ in range(NF)]
    ),
)
def _gather5(i0, i1, i2, i3, i4, t0, t1, t2, t3, t4, out_hbm,
             x0, x1, x2, x3, x4, r0, r1, r2, r3, r4, s0, s1, s2, s3, s4):
    wid = lax.axis_index("s") * _NC + lax.axis_index("c")
    base = wid * _BPW
    idxs = (i0, i1, i2, i3, i4)
    tabs = (t0, t1, t2, t3, t4)
    ivs = (x0, x1, x2, x3, x4)
    rvs = (r0, r1, r2, r3, r4)
    sems = (s0, s1, s2, s3, s4)
    for f in range(NF):
        pltpu.sync_copy(idxs[f].at[pl.ds(base, _BPW)], ivs[f])

    def _fire(f):
        tab, iv, rv, sem = tabs[f], ivs[f], rvs[f], sems[f]

        def body(g, carry):
            v = iv[pl.ds(g * 16, 16)]
            for j in range(16):
                row = v[j]
                pltpu.async_copy(tab.at[row], rv.at[g * 16 + j], sem)
            return carry

        lax.fori_loop(0, _BPW // 16, body, 0)

    def _drain(f):
        pltpu.make_async_copy(tabs[f].at[pl.ds(0, _BPW)], rvs[f], sems[f]).wait()
        pltpu.sync_copy(rvs[f], out_hbm.at[f, pl.ds(base, _BPW)])

    _fire(0)
    for f in range(1, NF):
        _fire(f)
        _drain(f - 1)
    _drain(NF - 1)


_BLK = 512


def _proj_body(emb_ref, wt_ref, b_ref, out_ref):
    e = emb_ref[...]
    x_c, x_cs, x_o, x_dp, x_bt = e[0], e[1], e[2], e[3], e[4]
    wt = wt_ref[...]
    bias = b_ref[...]
    y_c = jnp.dot(x_c, wt[0], preferred_element_type=jnp.float32) + bias[0] + x_cs
    y_o = jnp.dot(x_o, wt[1], preferred_element_type=jnp.float32) + bias[1] + x_dp
    y_dp = jnp.dot(x_dp, wt[2], preferred_element_type=jnp.float32) + bias[2] + x_bt
    out_ref[...] = jnp.concatenate([y_c, x_cs, y_o, y_dp, x_bt], axis=-1)


_proj = pl.pallas_call(
    _proj_body,
    grid=(B // _BLK,),
    in_specs=[
        pl.BlockSpec((NF, _BLK, D), lambda i: (0, i, 0)),
        pl.BlockSpec((3, D, D), lambda i: (0, 0, 0)),
        pl.BlockSpec((3, D), lambda i: (0, 0)),
    ],
    out_specs=pl.BlockSpec((_BLK, NF * D), lambda i: (i, 0)),
    out_shape=jax.ShapeDtypeStruct((B, NF * D), jnp.float32),
)


def kernel(campaignid, campaignsetid, offerid, demand_pkgname, business_type,
           table_campaignid, table_campaignsetid, table_offerid,
           table_demand_pkgname, table_business_type,
           W_campaignid, b_campaignid, W_offerid, b_offerid,
           W_demand_pkgname, b_demand_pkgname):
    idx = [x.astype(jnp.int32) for x in
           (campaignid, campaignsetid, offerid, demand_pkgname, business_type)]
    emb = _gather5(idx[0], idx[1], idx[2], idx[3], idx[4],
                   table_campaignid, table_campaignsetid, table_offerid,
                   table_demand_pkgname, table_business_type)
    wt = jnp.stack([W_campaignid.T, W_offerid.T, W_demand_pkgname.T])
    bias = jnp.stack([b_campaignid, b_offerid, b_demand_pkgname])
    return _proj(emb, wt, bias)


# speedup vs baseline: 2.5622x; 2.5622x over previous
import functools

import jax
import jax.numpy as jnp
from jax import lax
from jax.experimental import pallas as pl
from jax.experimental.pallas import tpu as pltpu
from jax.experimental.pallas import tpu_sc as plsc

D = 64
B = 4096
NF = 5
_STRIPE = 128
_NSB = 4

_info = plsc.get_sparse_core_info()
_NC = _info.num_cores
_NS = _info.num_subcores
_NW = _NC * _NS
_BPW = B // _NW

_mesh = plsc.VectorSubcoreMesh(core_axis_name="c", subcore_axis_name="s")


@functools.partial(
    pl.kernel,
    mesh=_mesh,
    compiler_params=pltpu.CompilerParams(needs_layout_passes=False),
    out_type=jax.ShapeDtypeStruct((NF, B, D), jnp.float32),
    scratch_types=(
        [pltpu.VMEM((_BPW,), jnp.int32) for _ in range(NF)]
        + [pltpu.VMEM((_BPW, D), jnp.float32) for _ in range(NF)]
        + [pltpu.VMEM((D, _STRIPE), jnp.float32) for _ in range(_NSB)]
        + [pltpu.SemaphoreType.DMA for _ in range(NF)]
        + [pltpu.SemaphoreType.DMA for _ in range(_NSB)]
    ),
)
def _gather5(i0, i1, i2, i3, i4, t0, t1, t2t, t3, t4, out_hbm,
             x0, x1, x2, x3, x4, r0, r1, r2, r3, r4,
             sb0, sb1, sb2, sb3,
             s0, s1, s2, s3, s4, q0, q1, q2, q3):
    wid = lax.axis_index("s") * _NC + lax.axis_index("c")
    base = wid * _BPW
    idxs = (i0, i1, i2, i3, i4)
    tabs = (t0, t1, None, t3, t4)
    ivs = (x0, x1, x2, x3, x4)
    rvs = (r0, r1, r2, r3, r4)
    sems = (s0, s1, s2, s3, s4)
    sbufs = (sb0, sb1, sb2, sb3)
    qsems = (q0, q1, q2, q3)
    for f in range(NF):
        pltpu.sync_copy(idxs[f].at[pl.ds(base, _BPW)], ivs[f])

    def _fire_rows(f):
        tab, iv, rv, sem = tabs[f], ivs[f], rvs[f], sems[f]

        def body(g, carry):
            v = iv[pl.ds(g * 16, 16)]
            for j in range(16):
                row = v[j]
                pltpu.async_copy(tab.at[row], rv.at[g * 16 + j], sem)
            return carry

        lax.fori_loop(0, _BPW // 16, body, 0)

    def _drain_rows(f):
        pltpu.make_async_copy(out_hbm.at[f, pl.ds(base, _BPW)], rvs[f],
                              sems[f]).wait()
        pltpu.sync_copy(rvs[f], out_hbm.at[f, pl.ds(base, _BPW)])

    for f in (0, 1, 3, 4):
        _fire_rows(f)

    iv2, rv2 = ivs[2], rvs[2]
    jvecs = [lax.iota(jnp.int32, 16) + 16 * k for k in range(4)]

    def _extract(lane, buf, i):
        lvec = jnp.full((16,), lane, dtype=jnp.int32)
        for k in range(4):
            col = plsc.load_gather(buf, [jvecs[k], lvec])
            rv2[i, pl.ds(k * 16, 16)] = col

    def _stripe_body(g, carry):
        v = iv2[pl.ds(g * 16, 16)]
        pend = []
        for j in range(16):
            row = v[j]
            base_lane = pl.multiple_of((row // _STRIPE) * _STRIPE, _STRIPE)
            lane = row - base_lane
            nb = j % _NSB
            if j >= _NSB:
                prow, plane, pcopy = pend[j - _NSB]
                pcopy.wait()
                _extract(plane, sbufs[nb], g * 16 + (j - _NSB))
            cp = pltpu.async_copy(
                t2t.at[:, pl.ds(base_lane, _STRIPE)], sbufs[nb], qsems[nb])
            pend.append((row, lane, cp))
        for j in range(16 - _NSB, 16):
            prow, plane, pcopy = pend[j]
            pcopy.wait()
            _extract(plane, sbufs[j % _NSB], g * 16 + j)
        return carry

    lax.fori_loop(0, _BPW // 16, _stripe_body, 0)
    pltpu.sync_copy(rv2, out_hbm.at[2, pl.ds(base, _BPW)])

    for f in (0, 1, 3, 4):
        _drain_rows(f)


_BLK = 512


def _proj_body(emb_ref, wt_ref, b_ref, out_ref):
    e = emb_ref[...]
    x_c, x_cs, x_o, x_dp, x_bt = e[0], e[1], e[2], e[3], e[4]
    wt = wt_ref[...]
    bias = b_ref[...]
    y_c = jnp.dot(x_c, wt[0], preferred_element_type=jnp.float32) + bias[0] + x_cs
    y_o = jnp.dot(x_o, wt[1], preferred_element_type=jnp.float32) + bias[1] + x_dp
    y_dp = jnp.dot(x_dp, wt[2], preferred_element_type=jnp.float32) + bias[2] + x_bt
    out_ref[...] = jnp.concatenate([y_c, x_cs, y_o, y_dp, x_bt], axis=-1)


_proj = pl.pallas_call(
    _proj_body,
    grid=(B // _BLK,),
    in_specs=[
        pl.BlockSpec((NF, _BLK, D), lambda i: (0, i, 0)),
        pl.BlockSpec((3, D, D), lambda i: (0, 0, 0)),
        pl.BlockSpec((3, D), lambda i: (0, 0)),
    ],
    out_specs=pl.BlockSpec((_BLK, NF * D), lambda i: (i, 0)),
    out_shape=jax.ShapeDtypeStruct((B, NF * D), jnp.float32),
)


def kernel(campaignid, campaignsetid, offerid, demand_pkgname, business_type,
           table_campaignid, table_campaignsetid, table_offerid,
           table_demand_pkgname, table_business_type,
           W_campaignid, b_campaignid, W_offerid, b_offerid,
           W_demand_pkgname, b_demand_pkgname):
    idx = [x.astype(jnp.int32) for x in
           (campaignid, campaignsetid, offerid, demand_pkgname, business_type)]
    emb = _gather5(idx[0], idx[1], idx[2], idx[3], idx[4],
                   table_campaignid, table_campaignsetid, table_offerid.T,
                   table_demand_pkgname, table_business_type)
    wt = jnp.stack([W_campaignid.T, W_offerid.T, W_demand_pkgname.T])
    bias = jnp.stack([b_campaignid, b_offerid, b_demand_pkgname])
    return _proj(emb, wt, bias)
